# baseline (speedup 1.0000x reference)
import jax
import jax.numpy as jnp
from jax import lax
from jax.experimental import pallas as pl
from jax.experimental.pallas import tpu as pltpu
from jax.experimental.pallas import tpu_sc as plsc

_VOCAB = 1000000
_CTX = 200
_D = 64
_B = 1024
_S = 200

_NW = 32
_ROWS = _B * _S
_PER_W = _ROWS // _NW
_CHUNK = 128
_NCH = _PER_W // _CHUNK
_NBUF = 2
_LANES = 16
_PK = 2 * _D


def _body(xf_hbm, tok_hbm, pos_hbm, out_hbm, x_v, idxp_v, pos_v, rows_v,
          outb_v, g0, g1, o0, o1):
    gsems = [g0, g1]
    osems = [o0, o1]
    wid = lax.axis_index("s") * 2 + lax.axis_index("c")
    base = wid * _PER_W

    pltpu.sync_copy(xf_hbm.at[pl.ds(base, _PER_W)], x_v)
    pltpu.sync_copy(pos_hbm, pos_v)

    def idx_step(g, carry):
        v = x_v[pl.ds(g * _LANES, _LANES)]
        idxp_v[pl.ds(g * _LANES, _LANES)] = lax.shift_right_logical(
            v, jnp.int32(1))
        return carry

    lax.fori_loop(0, _PER_W // _LANES, idx_step, 0, unroll=4)

    def fire_gather(c, b):
        return pltpu.async_copy(
            tok_hbm.at[idxp_v.at[pl.ds(c * _CHUNK, _CHUNK)]],
            rows_v.at[b], gsems[b])

    fire_gather(0, 0)
    fire_gather(1, 1)

    def round_step(g, carry):
        for b in range(_NBUF):
            c = g * _NBUF + b
            pltpu.make_async_copy(tok_hbm.at[pl.ds(0, _CHUNK)],
                                  rows_v.at[b], gsems[b]).wait()
            @pl.when(g > 0)
            def _():
                pltpu.make_async_copy(
                    outb_v.at[b], out_hbm.at[pl.ds(0, _CHUNK)],
                    osems[b]).wait()

            pstart = lax.rem(c * _CHUNK, _CTX)

            def ex_step(j, carry2):
                xh = x_v[pl.ds(c * _CHUNK + j * _LANES, _LANES)]
                hvec = lax.shift_left(
                    lax.bitwise_and(xh, jnp.int32(1)), jnp.int32(6))
                for i in range(_LANES):
                    r = j * _LANES + i
                    h = hvec[i]
                    rp = pstart + r
                    rp = lax.select(rp >= _CTX, rp - _CTX, rp)
                    for k in range(_D // _LANES):
                        v = rows_v[b, r, pl.ds(h + k * _LANES, _LANES)]
                        pv = pos_v[rp, pl.ds(k * _LANES, _LANES)]
                        outb_v[b, r, pl.ds(k * _LANES, _LANES)] = v + pv
                return carry2

            lax.fori_loop(0, _CHUNK // _LANES, ex_step, 0)

            pltpu.async_copy(
                outb_v.at[b],
                out_hbm.at[pl.ds(base + c * _CHUNK, _CHUNK)], osems[b])

            @pl.when(c + _NBUF < _NCH)
            def _():
                fire_gather(c + _NBUF, b)

        return carry

    lax.fori_loop(0, _NCH // _NBUF, round_step, 0)

    for b in range(_NBUF):
        pltpu.make_async_copy(outb_v.at[b], out_hbm.at[pl.ds(0, _CHUNK)],
                              osems[b]).wait()


@jax.jit
def kernel(x, token_table, pos_table):
    xf = x.reshape(_ROWS).astype(jnp.int32)
    tok2 = token_table.reshape(_VOCAB // 2, _PK)
    mesh = plsc.VectorSubcoreMesh(core_axis_name="c", subcore_axis_name="s")
    out = pl.kernel(
        _body,
        out_type=jax.ShapeDtypeStruct((_ROWS, _D), jnp.float32),
        mesh=mesh,
        compiler_params=pltpu.CompilerParams(use_tc_tiling_on_sc=False),
        scratch_types=[
            pltpu.VMEM((_PER_W,), jnp.int32),
            pltpu.VMEM((_PER_W,), jnp.int32),
            pltpu.VMEM((_CTX, _D), jnp.float32),
            pltpu.VMEM((_NBUF, _CHUNK, _PK), jnp.float32),
            pltpu.VMEM((_NBUF, _CHUNK, _D), jnp.float32),
        ] + [pltpu.SemaphoreType.DMA] * (2 * _NBUF),
    )(xf, tok2, pos_table)
    return out.reshape(_B, _S, _D)


# speedup vs baseline: 1.1151x; 1.1151x over previous
import jax
import jax.numpy as jnp
from jax import lax
from jax.experimental import pallas as pl
from jax.experimental.pallas import tpu as pltpu
from jax.experimental.pallas import tpu_sc as plsc

_VOCAB = 1000000
_CTX = 200
_D = 64
_B = 1024
_S = 200

_NW = 32
_ROWS = _B * _S
_PER_W = _ROWS // _NW
_GCHUNK = 100
_CHUNK = 200
_NIDX = _PER_W // _GCHUNK
_NCH = _PER_W // _CHUNK
_NBUF = 8
_ROUNDS = _NCH // _NBUF
_LANES = 16


def _add_pos(rows_v, pos_v, b):
    def add_step(r, carry):
        for k in range(_D // _LANES):
            vec = pos_v[r, pl.ds(k * _LANES, _LANES)]
            plsc.addupdate(rows_v.at[b, r, pl.ds(k * _LANES, _LANES)], vec)
        return carry

    lax.fori_loop(0, _CHUNK, add_step, 0, unroll=2)


def _body(x_hbm, tok_hbm, pos_hbm, out_hbm, idx_v, pos_v, rows_v,
          g0, g1, g2, g3, g4, g5, g6, g7, osem):
    gsems = [g0, g1, g2, g3, g4, g5, g6, g7]
    wid = lax.axis_index("s") * 2 + lax.axis_index("c")
    base = wid * _PER_W

    pltpu.sync_copy(x_hbm.at[pl.ds(wid * _NIDX, _NIDX)], idx_v)
    pltpu.sync_copy(pos_hbm, pos_v)

    def round_step(g, carry):
        c0 = g * _NBUF

        descs = []
        for b in range(_NBUF):
            c = c0 + b
            d0 = pltpu.async_copy(tok_hbm.at[idx_v.at[2 * c]],
                                  rows_v.at[b, pl.ds(0, _GCHUNK)], gsems[b])
            d1 = pltpu.async_copy(tok_hbm.at[idx_v.at[2 * c + 1]],
                                  rows_v.at[b, pl.ds(_GCHUNK, _GCHUNK)],
                                  gsems[b])
            descs.append((d0, d1))

        sdescs = []
        for b in range(_NBUF):
            c = c0 + b
            descs[b][0].wait()
            descs[b][1].wait()
            _add_pos(rows_v, pos_v, b)
            sdescs.append(pltpu.async_copy(
                rows_v.at[b],
                out_hbm.at[pl.ds(base + c * _CHUNK, _CHUNK)], osem))

        for b in range(_NBUF):
            sdescs[b].wait()
        return carry

    lax.fori_loop(0, _ROUNDS, round_step, 0)


@jax.jit
def kernel(x, token_table, pos_table):
    x_flat = x.reshape(_NW * _NIDX, _GCHUNK).astype(jnp.int32)
    mesh = plsc.VectorSubcoreMesh(core_axis_name="c", subcore_axis_name="s")
    out = pl.kernel(
        _body,
        out_type=jax.ShapeDtypeStruct((_ROWS, _D), jnp.float32),
        mesh=mesh,
        compiler_params=pltpu.CompilerParams(use_tc_tiling_on_sc=False),
        scratch_types=[
            pltpu.VMEM((_NIDX, _GCHUNK), jnp.int32),
            pltpu.VMEM((_CTX, _D), jnp.float32),
            pltpu.VMEM((_NBUF, _CHUNK, _D), jnp.float32),
        ] + [pltpu.SemaphoreType.DMA] * (_NBUF + 1),
    )(x_flat, token_table, pos_table)
    return out.reshape(_B, _S, _D)
